# baseline (speedup 1.0000x reference)
import functools

import jax
import jax.numpy as jnp
import numpy as np
from jax import lax
from jax.experimental import pallas as pl
from jax.experimental.pallas import tpu as pltpu
from jax.experimental.pallas import tpu_sc as plsc

B = 2
N = 8192
CIN = 4
WIDTH = 64
OUT = 128
STRIDE = 4
NS = 32
R1 = 0.1
R2 = 0.2
MID = 512
M = N // STRIDE

MB = 256
NBLK = 1024


def _stem_body(f_ref, w_ref, b_ref, o_ref):
    x = f_ref[0]
    w = w_ref[...]
    o_ref[0] = jnp.dot(x, w, preferred_element_type=jnp.float32) + b_ref[...]


def _stem(f_t, w_t, b2):
    return pl.pallas_call(
        _stem_body,
        grid=(B, N // NBLK),
        in_specs=[
            pl.BlockSpec((1, NBLK, CIN), lambda b, i: (b, i, 0)),
            pl.BlockSpec((CIN, WIDTH), lambda b, i: (0, 0)),
            pl.BlockSpec((1, WIDTH), lambda b, i: (0, 0)),
        ],
        out_specs=pl.BlockSpec((1, NBLK, WIDTH), lambda b, i: (b, i, 0)),
        out_shape=jax.ShapeDtypeStruct((B, N, WIDTH), jnp.float32),
    )(f_t, w_t, b2)


def _stage1_body(dp_ref, xf_ref, wp_ref, wf_ref, g_ref, b_ref, o_ref):
    dp = dp_ref[...].reshape(MB * NS, 3)
    xf = xf_ref[...].reshape(MB * NS, WIDTH)
    y = jnp.dot(dp, wp_ref[...], preferred_element_type=jnp.float32)
    y = y + jnp.dot(xf, wf_ref[...], preferred_element_type=jnp.float32)
    y = y * g_ref[...] + b_ref[...]
    y = jnp.maximum(y, 0.0)
    o_ref[0] = jnp.max(y.reshape(MB, NS, OUT), axis=1)


def _stage1(dp, xf, wp, wf, g2, b2):
    return pl.pallas_call(
        _stage1_body,
        grid=(B, M // MB),
        in_specs=[
            pl.BlockSpec((1, MB, NS, 3), lambda b, i: (b, i, 0, 0)),
            pl.BlockSpec((1, MB, NS, WIDTH), lambda b, i: (b, i, 0, 0)),
            pl.BlockSpec((3, OUT), lambda b, i: (0, 0)),
            pl.BlockSpec((WIDTH, OUT), lambda b, i: (0, 0)),
            pl.BlockSpec((1, OUT), lambda b, i: (0, 0)),
            pl.BlockSpec((1, OUT), lambda b, i: (0, 0)),
        ],
        out_specs=pl.BlockSpec((1, MB, OUT), lambda b, i: (b, i, 0)),
        out_shape=jax.ShapeDtypeStruct((B, M, OUT), jnp.float32),
    )(dp, xf, wp, wf, g2, b2)


def _stage2_body(dp_ref, xf_ref, id_ref, wp_ref, wf_ref, gla_ref, bla_ref,
                 w1_ref, g1_ref, b1_ref, w2_ref, g2_ref, b2_ref, o_ref):
    dp = dp_ref[...].reshape(MB * NS, 3)
    xf = xf_ref[...].reshape(MB * NS, OUT)
    y = jnp.dot(dp, wp_ref[...], preferred_element_type=jnp.float32)
    y = y + jnp.dot(xf, wf_ref[...], preferred_element_type=jnp.float32)
    y = y * gla_ref[...] + bla_ref[...]
    y = jnp.maximum(y, 0.0)
    f2 = jnp.max(y.reshape(MB, NS, OUT), axis=1)
    h = jnp.dot(f2, w1_ref[...], preferred_element_type=jnp.float32)
    h = jnp.maximum(h * g1_ref[...] + b1_ref[...], 0.0)
    o = jnp.dot(h, w2_ref[...], preferred_element_type=jnp.float32)
    o = o * g2_ref[...] + b2_ref[...]
    o_ref[0] = jnp.maximum(o + id_ref[0], 0.0)


def _stage2(dp, xf, ident, wp, wf, gla2, bla2, w1t, g12, b12, w2t, g22, b22):
    return pl.pallas_call(
        _stage2_body,
        grid=(B, M // MB),
        in_specs=[
            pl.BlockSpec((1, MB, NS, 3), lambda b, i: (b, i, 0, 0)),
            pl.BlockSpec((1, MB, NS, OUT), lambda b, i: (b, i, 0, 0)),
            pl.BlockSpec((1, MB, OUT), lambda b, i: (b, i, 0)),
            pl.BlockSpec((3, OUT), lambda b, i: (0, 0)),
            pl.BlockSpec((OUT, OUT), lambda b, i: (0, 0)),
            pl.BlockSpec((1, OUT), lambda b, i: (0, 0)),
            pl.BlockSpec((1, OUT), lambda b, i: (0, 0)),
            pl.BlockSpec((OUT, MID), lambda b, i: (0, 0)),
            pl.BlockSpec((1, MID), lambda b, i: (0, 0)),
            pl.BlockSpec((1, MID), lambda b, i: (0, 0)),
            pl.BlockSpec((MID, OUT), lambda b, i: (0, 0)),
            pl.BlockSpec((1, OUT), lambda b, i: (0, 0)),
            pl.BlockSpec((1, OUT), lambda b, i: (0, 0)),
        ],
        out_specs=pl.BlockSpec((1, MB, OUT), lambda b, i: (b, i, 0)),
        out_shape=jax.ShapeDtypeStruct((B, M, OUT), jnp.float32),
    )(dp, xf, ident, wp, wf, gla2, bla2, w1t, g12, b12, w2t, g22, b22)


BIGF = 1e30
BIGI = 2**30

NTILES = 32
QPT = (B * M) // NTILES
CAP = 512


def _make_sc_ballquery(ncand, r2):
    mesh = plsc.VectorSubcoreMesh(core_axis_name="c", subcore_axis_name="s")

    @functools.partial(
        pl.kernel, mesh=mesh,
        out_type=jax.ShapeDtypeStruct((B * M * NS,), jnp.int32),
        scratch_types=[
            pltpu.VMEM((ncand,), jnp.float32),
            pltpu.VMEM((ncand,), jnp.float32),
            pltpu.VMEM((ncand,), jnp.float32),
            pltpu.VMEM((QPT,), jnp.float32),
            pltpu.VMEM((QPT,), jnp.float32),
            pltpu.VMEM((QPT,), jnp.float32),
            pltpu.VMEM((CAP,), jnp.float32),
            pltpu.VMEM((CAP,), jnp.int32),
            pltpu.VMEM((QPT * NS,), jnp.int32),
            pltpu.VMEM((ncand,), jnp.float32),
        ],
        compiler_params=pltpu.CompilerParams(needs_layout_passes=False),
    )
    def k(cx, cy, cz, qx, qy, qz, o_hbm,
          xs, ys, zs, qxv, qyv, qzv, keyb, idxb, outbuf, s2s):
        cid = lax.axis_index("c")
        sid = lax.axis_index("s")
        wid = sid * 2 + cid
        b = wid // (NTILES // B)
        pltpu.sync_copy(cx.at[b], xs)
        pltpu.sync_copy(cy.at[b], ys)
        pltpu.sync_copy(cz.at[b], zs)
        base_q = wid * QPT
        pltpu.sync_copy(qx.at[pl.ds(base_q, QPT)], qxv)
        pltpu.sync_copy(qy.at[pl.ds(base_q, QPT)], qyv)
        pltpu.sync_copy(qz.at[pl.ds(base_q, QPT)], qzv)
        iota16 = lax.iota(jnp.int32, 16)
        zeros16 = jnp.zeros((16,), jnp.int32)
        bigv = jnp.full((16,), BIGF, jnp.float32)

        def _rbf16(v):
            u = plsc.bitcast(v, jnp.int32)
            u = u + 0x7FFF + ((u >> 16) & 1)
            u = u & jnp.int32(-65536)
            return plsc.bitcast(u, jnp.float32)

        def prep(t, _):
            xv = xs[pl.ds(t * 16, 16)]
            yv = ys[pl.ds(t * 16, 16)]
            zv = zs[pl.ds(t * 16, 16)]
            s2s[pl.ds(t * 16, 16)] = (xv * xv + yv * yv) + zv * zv
            xs[pl.ds(t * 16, 16)] = _rbf16(xv)
            ys[pl.ds(t * 16, 16)] = _rbf16(yv)
            zs[pl.ds(t * 16, 16)] = _rbf16(zv)
            return 0

        lax.fori_loop(0, ncand // 16, prep, 0)

        def per_query(i, _):
            tq = (i // 16) * 16
            lane = zeros16 + (i - tq)
            qxb = qxv[pl.ds(tq, 16)][lane]
            qyb = qyv[pl.ds(tq, 16)][lane]
            qzb = qzv[pl.ds(tq, 16)][lane]
            q2b = (qxb * qxb + qyb * qyb) + qzb * qzb
            qxr = _rbf16(qxb)
            qyr = _rbf16(qyb)
            qzr = _rbf16(qzb)

            def body(t, cntv):
                dot = (xs[pl.ds(t * 16, 16)] * qxr
                       + ys[pl.ds(t * 16, 16)] * qyr
                       + zs[pl.ds(t * 16, 16)] * qzr)
                d2 = (q2b + s2s[pl.ds(t * 16, 16)]) - 2.0 * dot
                mask = d2 <= r2
                pos = plsc.cumsum(mask.astype(jnp.int32)) + cntv - 1
                pos = jnp.minimum(pos, CAP - 1)
                plsc.store_scatter(keyb, [pos], d2, mask=mask)
                plsc.store_scatter(idxb, [pos], iota16 + t * 16, mask=mask)
                return cntv + plsc.all_reduce_population_count(mask)

            cntv = lax.fori_loop(0, ncand // 16, body, zeros16)
            cnt = jnp.minimum(jnp.max(cntv), CAP)
            keyb[pl.ds(jnp.minimum(cnt, CAP - 16), 16)] = bigv
            nv = (cnt + 15) // 16

            def sel(t, carry):
                a0k, a0v, a1k, a1v = carry
                ck = keyb[pl.ds(t * 16, 16)]
                cv = idxb[pl.ds(t * 16, 16)]
                ck, cv = plsc.sort_key_val(ck, cv)
                ckr = lax.rev(ck, (0,))
                cvr = lax.rev(cv, (0,))
                m = a1k <= ckr
                lk = jnp.where(m, a1k, ckr)
                lv = jnp.where(m, a1v, cvr)
                lk, lv = plsc.sort_key_val(lk, lv)
                lkr = lax.rev(lk, (0,))
                lvr = lax.rev(lv, (0,))
                m2 = a0k <= lkr
                n0k = jnp.where(m2, a0k, lkr)
                n0v = jnp.where(m2, a0v, lvr)
                n1k = jnp.where(m2, lkr, a0k)
                n1v = jnp.where(m2, lvr, a0v)
                n0k, n0v = plsc.sort_key_val(n0k, n0v)
                n1k, n1v = plsc.sort_key_val(n1k, n1v)
                return n0k, n0v, n1k, n1v

            a0k, a0v, a1k, a1v = lax.fori_loop(
                0, nv, sel, (bigv, zeros16, bigv, zeros16))
            pad = a0v[zeros16]
            out0 = jnp.where(iota16 < cntv, a0v, pad)
            out1 = jnp.where(iota16 + 16 < cntv, a1v, pad)
            outbuf[pl.ds(i * NS, 16)] = out0
            outbuf[pl.ds(i * NS + 16, 16)] = out1
            return 0

        lax.fori_loop(0, QPT, per_query, 0)
        pltpu.sync_copy(outbuf, o_hbm.at[pl.ds(base_q * NS, QPT * NS)])

    return k


def _sc_ball(q, s, radius, ncand):
    k = _make_sc_ballquery(ncand, radius * radius)
    cx = s[..., 0] + 0.0
    cy = s[..., 1] + 0.0
    cz = s[..., 2] + 0.0
    qx = q[..., 0].reshape(B * M)
    qy = q[..., 1].reshape(B * M)
    qz = q[..., 2].reshape(B * M)
    nb = k(cx, cy, cz, qx, qy, qz)
    return nb.reshape(B, M, NS)


def _select_body(ncand, r2, q_ref, st_ref, o_ref):
    q = q_ref[0]
    st = st_ref[0]
    q2 = jnp.sum(q * q, axis=1, keepdims=True)
    s2 = jnp.sum(st * st, axis=0, keepdims=True)
    d2 = q2 + s2 - 2.0 * jnp.dot(q, st, preferred_element_type=jnp.float32)
    iota = jax.lax.broadcasted_iota(jnp.int32, (MB, ncand), 1)
    mfull = jnp.min(d2, axis=1, keepdims=True)
    nearest = jnp.min(jnp.where(d2 <= mfull, iota, BIGI), axis=1, keepdims=True)
    cur = jnp.where(d2 <= r2, d2, BIGF)
    for k in range(NS):
        m = jnp.min(cur, axis=1, keepdims=True)
        a = jnp.min(jnp.where(cur <= m, iota, BIGI), axis=1, keepdims=True)
        o_ref[0, :, k:k + 1] = jnp.where(m < BIGF, a, nearest)
        cur = jnp.where(iota == a, BIGF, cur)


def _ball_select(q, st, radius, ncand):
    return pl.pallas_call(
        functools.partial(_select_body, ncand, radius * radius),
        grid=(B, M // MB),
        in_specs=[
            pl.BlockSpec((1, MB, 3), lambda b, i: (b, i, 0)),
            pl.BlockSpec((1, 3, ncand), lambda b, i: (b, 0, 0)),
        ],
        out_specs=pl.BlockSpec((1, MB, NS), lambda b, i: (b, i, 0)),
        out_shape=jax.ShapeDtypeStruct((B, M, NS), jnp.int32),
    )(q, st)


def _sample_idx():
    skey = jax.random.key(42)
    return jnp.stack([
        jax.random.choice(jax.random.fold_in(skey, i), N, shape=(M,), replace=False)
        for i in range(B)
    ])


def kernel(p, f, w_stem, b_stem, w_sa, g_sa, b_sa, w_la, g_la, b_la,
           w_pw1, g_pw1, b_pw1, w_pw2, g_pw2, b_pw2):
    f0 = _stem(jnp.transpose(f, (0, 2, 1)), w_stem.T, b_stem[None, :])

    idx_s = _sample_idx()
    new_p = jax.vmap(lambda pb, ib: pb[ib])(p, idx_s)

    nb = _sc_ball(new_p, p, R1, N)
    pj = jax.vmap(lambda pb, ib: pb[ib])(p, nb)
    dp = pj - new_p[:, :, None, :]
    f0_cf = jnp.transpose(f0, (0, 2, 1))
    xf = jnp.transpose(jax.vmap(lambda fb, ib: fb[:, ib])(f0_cf, nb), (0, 2, 3, 1))

    f1 = _stage1(dp, xf, w_sa[:, :3].T, w_sa[:, 3:].T,
                 g_sa[None, :], b_sa[None, :])

    nb2 = _sc_ball(new_p, new_p, R2, M)
    pj2 = jax.vmap(lambda pb, ib: pb[ib])(new_p, nb2)
    dp2 = pj2 - new_p[:, :, None, :]
    f1_cf = jnp.transpose(f1, (0, 2, 1))
    xf2 = jnp.transpose(jax.vmap(lambda fb, ib: fb[:, ib])(f1_cf, nb2), (0, 2, 3, 1))

    out = _stage2(dp2, xf2, f1, w_la[:, :3].T, w_la[:, 3:].T,
                  g_la[None, :], b_la[None, :],
                  w_pw1.T, g_pw1[None, :], b_pw1[None, :],
                  w_pw2.T, g_pw2[None, :], b_pw2[None, :])
    return jnp.transpose(out, (0, 2, 1))


# speedup vs baseline: 1.9205x; 1.9205x over previous
import functools

import jax
import jax.numpy as jnp
import numpy as np
from jax import lax
from jax.experimental import pallas as pl
from jax.experimental.pallas import tpu as pltpu
from jax.experimental.pallas import tpu_sc as plsc

B = 2
N = 8192
CIN = 4
WIDTH = 64
OUT = 128
STRIDE = 4
NS = 32
R1 = 0.1
R2 = 0.2
MID = 512
M = N // STRIDE

MB = 256
NBLK = 1024


def _stem_body(f_ref, w_ref, b_ref, o_ref):
    x = f_ref[0]
    w = w_ref[...]
    o_ref[0] = jnp.dot(x, w, preferred_element_type=jnp.float32) + b_ref[...]


def _stem(f_t, w_t, b2):
    return pl.pallas_call(
        _stem_body,
        grid=(B, N // NBLK),
        in_specs=[
            pl.BlockSpec((1, NBLK, CIN), lambda b, i: (b, i, 0)),
            pl.BlockSpec((CIN, WIDTH), lambda b, i: (0, 0)),
            pl.BlockSpec((1, WIDTH), lambda b, i: (0, 0)),
        ],
        out_specs=pl.BlockSpec((1, NBLK, WIDTH), lambda b, i: (b, i, 0)),
        out_shape=jax.ShapeDtypeStruct((B, N, WIDTH), jnp.float32),
    )(f_t, w_t, b2)


def _stage1_body(dp_ref, xf_ref, wp_ref, wf_ref, g_ref, b_ref, o_ref):
    dp = dp_ref[...].reshape(MB * NS, 3)
    xf = xf_ref[...].reshape(MB * NS, 2 * WIDTH)
    y = jnp.dot(dp, wp_ref[...], preferred_element_type=jnp.float32)
    y = y + jnp.dot(xf, wf_ref[...], preferred_element_type=jnp.float32)
    y = y * g_ref[...] + b_ref[...]
    y = jnp.maximum(y, 0.0)
    o_ref[0] = jnp.max(y.reshape(MB, NS, OUT), axis=1)


def _stage1(dp, xf, wp, wf, g2, b2):
    return pl.pallas_call(
        _stage1_body,
        grid=(B, M // MB),
        in_specs=[
            pl.BlockSpec((1, MB, NS, 3), lambda b, i: (b, i, 0, 0)),
            pl.BlockSpec((1, MB, NS, 2 * WIDTH), lambda b, i: (b, i, 0, 0)),
            pl.BlockSpec((3, OUT), lambda b, i: (0, 0)),
            pl.BlockSpec((2 * WIDTH, OUT), lambda b, i: (0, 0)),
            pl.BlockSpec((1, OUT), lambda b, i: (0, 0)),
            pl.BlockSpec((1, OUT), lambda b, i: (0, 0)),
        ],
        out_specs=pl.BlockSpec((1, MB, OUT), lambda b, i: (b, i, 0)),
        out_shape=jax.ShapeDtypeStruct((B, M, OUT), jnp.float32),
    )(dp, xf, wp, wf, g2, b2)


def _stage2_body(dp_ref, xf_ref, id_ref, wp_ref, wf_ref, gla_ref, bla_ref,
                 w1_ref, g1_ref, b1_ref, w2_ref, g2_ref, b2_ref, o_ref):
    dp = dp_ref[...].reshape(MB * NS, 3)
    xf = xf_ref[...].reshape(MB * NS, OUT)
    y = jnp.dot(dp, wp_ref[...], preferred_element_type=jnp.float32)
    y = y + jnp.dot(xf, wf_ref[...], preferred_element_type=jnp.float32)
    y = y * gla_ref[...] + bla_ref[...]
    y = jnp.maximum(y, 0.0)
    f2 = jnp.max(y.reshape(MB, NS, OUT), axis=1)
    h = jnp.dot(f2, w1_ref[...], preferred_element_type=jnp.float32)
    h = jnp.maximum(h * g1_ref[...] + b1_ref[...], 0.0)
    o = jnp.dot(h, w2_ref[...], preferred_element_type=jnp.float32)
    o = o * g2_ref[...] + b2_ref[...]
    o_ref[0] = jnp.maximum(o + id_ref[0], 0.0)


def _stage2(dp, xf, ident, wp, wf, gla2, bla2, w1t, g12, b12, w2t, g22, b22):
    return pl.pallas_call(
        _stage2_body,
        grid=(B, M // MB),
        in_specs=[
            pl.BlockSpec((1, MB, NS, 3), lambda b, i: (b, i, 0, 0)),
            pl.BlockSpec((1, MB, NS, OUT), lambda b, i: (b, i, 0, 0)),
            pl.BlockSpec((1, MB, OUT), lambda b, i: (b, i, 0)),
            pl.BlockSpec((3, OUT), lambda b, i: (0, 0)),
            pl.BlockSpec((OUT, OUT), lambda b, i: (0, 0)),
            pl.BlockSpec((1, OUT), lambda b, i: (0, 0)),
            pl.BlockSpec((1, OUT), lambda b, i: (0, 0)),
            pl.BlockSpec((OUT, MID), lambda b, i: (0, 0)),
            pl.BlockSpec((1, MID), lambda b, i: (0, 0)),
            pl.BlockSpec((1, MID), lambda b, i: (0, 0)),
            pl.BlockSpec((MID, OUT), lambda b, i: (0, 0)),
            pl.BlockSpec((1, OUT), lambda b, i: (0, 0)),
            pl.BlockSpec((1, OUT), lambda b, i: (0, 0)),
        ],
        out_specs=pl.BlockSpec((1, MB, OUT), lambda b, i: (b, i, 0)),
        out_shape=jax.ShapeDtypeStruct((B, M, OUT), jnp.float32),
    )(dp, xf, ident, wp, wf, gla2, bla2, w1t, g12, b12, w2t, g22, b22)


BIGF = 1e30
BIGI = 2**30

NTILES = 32
QPT = (B * M) // NTILES
CAP = 512


def _make_sc_ballquery(ncand, r2, feat_dim=None):
    mesh = plsc.VectorSubcoreMesh(core_axis_name="c", subcore_axis_name="s")
    out_type = jax.ShapeDtypeStruct((B * M * NS,), jnp.int32)
    scratch = [
        pltpu.VMEM((ncand,), jnp.float32),
        pltpu.VMEM((ncand,), jnp.float32),
        pltpu.VMEM((ncand,), jnp.float32),
        pltpu.VMEM((QPT,), jnp.float32),
        pltpu.VMEM((QPT,), jnp.float32),
        pltpu.VMEM((QPT,), jnp.float32),
        pltpu.VMEM((CAP,), jnp.float32),
        pltpu.VMEM((CAP,), jnp.int32),
        pltpu.VMEM((QPT * NS,), jnp.int32),
        pltpu.VMEM((ncand,), jnp.float32),
    ]
    if feat_dim is not None:
        out_type = (out_type,
                    jax.ShapeDtypeStruct((B * M * NS, feat_dim), jnp.float32))
        scratch += [
            pltpu.VMEM((NS,), jnp.int32),
            pltpu.VMEM((NS, feat_dim), jnp.float32),
            pltpu.SemaphoreType.DMA,
        ]

    @functools.partial(
        pl.kernel, mesh=mesh, out_type=out_type, scratch_types=scratch,
        compiler_params=pltpu.CompilerParams(needs_layout_passes=False),
    )
    def k(*refs):
        if feat_dim is not None:
            (cx, cy, cz, qx, qy, qz, tab, o_hbm, xf_hbm,
             xs, ys, zs, qxv, qyv, qzv, keyb, idxb, outbuf, s2s,
             gidx, rows, dsem) = refs
        else:
            (cx, cy, cz, qx, qy, qz, o_hbm,
             xs, ys, zs, qxv, qyv, qzv, keyb, idxb, outbuf, s2s) = refs
        cid = lax.axis_index("c")
        sid = lax.axis_index("s")
        wid = sid * 2 + cid
        b = wid // (NTILES // B)
        pltpu.sync_copy(cx.at[b], xs)
        pltpu.sync_copy(cy.at[b], ys)
        pltpu.sync_copy(cz.at[b], zs)
        base_q = wid * QPT
        pltpu.sync_copy(qx.at[pl.ds(base_q, QPT)], qxv)
        pltpu.sync_copy(qy.at[pl.ds(base_q, QPT)], qyv)
        pltpu.sync_copy(qz.at[pl.ds(base_q, QPT)], qzv)
        iota16 = lax.iota(jnp.int32, 16)
        zeros16 = jnp.zeros((16,), jnp.int32)
        bigv = jnp.full((16,), BIGF, jnp.float32)

        def _rbf16(v):
            u = plsc.bitcast(v, jnp.int32)
            u = u + 0x7FFF + ((u >> 16) & 1)
            u = u & jnp.int32(-65536)
            return plsc.bitcast(u, jnp.float32)

        def prep(t, _):
            xv = xs[pl.ds(t * 16, 16)]
            yv = ys[pl.ds(t * 16, 16)]
            zv = zs[pl.ds(t * 16, 16)]
            s2s[pl.ds(t * 16, 16)] = (xv * xv + yv * yv) + zv * zv
            xs[pl.ds(t * 16, 16)] = _rbf16(xv)
            ys[pl.ds(t * 16, 16)] = _rbf16(yv)
            zs[pl.ds(t * 16, 16)] = _rbf16(zv)
            return 0

        lax.fori_loop(0, ncand // 16, prep, 0)

        def per_query(i, _):
            tq = (i // 16) * 16
            lane = zeros16 + (i - tq)
            qxb = qxv[pl.ds(tq, 16)][lane]
            qyb = qyv[pl.ds(tq, 16)][lane]
            qzb = qzv[pl.ds(tq, 16)][lane]
            q2b = (qxb * qxb + qyb * qyb) + qzb * qzb
            qxr = _rbf16(qxb)
            qyr = _rbf16(qyb)
            qzr = _rbf16(qzb)

            def body(t, cntv):
                dot = (xs[pl.ds(t * 16, 16)] * qxr
                       + ys[pl.ds(t * 16, 16)] * qyr
                       + zs[pl.ds(t * 16, 16)] * qzr)
                d2 = (q2b + s2s[pl.ds(t * 16, 16)]) - 2.0 * dot
                mask = d2 <= r2
                pos = plsc.cumsum(mask.astype(jnp.int32)) + cntv - 1
                pos = jnp.minimum(pos, CAP - 1)
                plsc.store_scatter(keyb, [pos], d2, mask=mask)
                plsc.store_scatter(idxb, [pos], iota16 + t * 16, mask=mask)
                return cntv + plsc.all_reduce_population_count(mask)

            cntv = lax.fori_loop(0, ncand // 16, body, zeros16)
            cnt = jnp.minimum(jnp.max(cntv), CAP)
            keyb[pl.ds(jnp.minimum(cnt, CAP - 16), 16)] = bigv
            nv = (cnt + 15) // 16

            def sel(t, carry):
                a0k, a0v, a1k, a1v = carry
                ck = keyb[pl.ds(t * 16, 16)]
                cv = idxb[pl.ds(t * 16, 16)]
                ck, cv = plsc.sort_key_val(ck, cv)
                ckr = lax.rev(ck, (0,))
                cvr = lax.rev(cv, (0,))
                m = a1k <= ckr
                lk = jnp.where(m, a1k, ckr)
                lv = jnp.where(m, a1v, cvr)
                lk, lv = plsc.sort_key_val(lk, lv)
                lkr = lax.rev(lk, (0,))
                lvr = lax.rev(lv, (0,))
                m2 = a0k <= lkr
                n0k = jnp.where(m2, a0k, lkr)
                n0v = jnp.where(m2, a0v, lvr)
                n1k = jnp.where(m2, lkr, a0k)
                n1v = jnp.where(m2, lvr, a0v)
                n0k, n0v = plsc.sort_key_val(n0k, n0v)
                n1k, n1v = plsc.sort_key_val(n1k, n1v)
                return n0k, n0v, n1k, n1v

            a0k, a0v, a1k, a1v = lax.fori_loop(
                0, nv, sel, (bigv, zeros16, bigv, zeros16))
            pad = a0v[zeros16]
            out0 = jnp.where(iota16 < cntv, a0v, pad)
            out1 = jnp.where(iota16 + 16 < cntv, a1v, pad)
            outbuf[pl.ds(i * NS, 16)] = out0
            outbuf[pl.ds(i * NS + 16, 16)] = out1
            if feat_dim is not None:
                gidx[pl.ds(0, 16)] = out0 + b * ncand
                gidx[pl.ds(16, 16)] = out1 + b * ncand
                pltpu.async_copy(tab.at[gidx], rows, dsem).wait()
                pltpu.sync_copy(
                    rows, xf_hbm.at[pl.ds((base_q + i) * NS, NS)])
            return 0

        lax.fori_loop(0, QPT, per_query, 0)
        pltpu.sync_copy(outbuf, o_hbm.at[pl.ds(base_q * NS, QPT * NS)])

    return k


def _make_sc_rowgather(feat_dim):
    mesh = plsc.VectorSubcoreMesh(core_axis_name="c", subcore_axis_name="s")
    NIDX = B * M * NS
    IPT = NIDX // NTILES

    @functools.partial(
        pl.kernel, mesh=mesh,
        out_type=jax.ShapeDtypeStruct((NIDX, feat_dim), jnp.float32),
        scratch_types=[
            pltpu.VMEM((IPT,), jnp.int32),
            pltpu.VMEM((NS, feat_dim), jnp.float32),
            pltpu.SemaphoreType.DMA,
        ],
        compiler_params=pltpu.CompilerParams(needs_layout_passes=False),
    )
    def k(idx_hbm, tab, o_hbm, idxv, rows, dsem):
        cid = lax.axis_index("c")
        sid = lax.axis_index("s")
        wid = sid * 2 + cid
        base = wid * IPT
        pltpu.sync_copy(idx_hbm.at[pl.ds(base, IPT)], idxv)

        def per_chunk(i, _):
            pltpu.async_copy(
                tab.at[idxv.at[pl.ds(i * NS, NS)]], rows, dsem).wait()
            pltpu.sync_copy(rows, o_hbm.at[pl.ds(base + i * NS, NS)])
            return 0

        lax.fori_loop(0, IPT // NS, per_chunk, 0)

    return k


def _sc_ball(q, s, radius, ncand, tab=None, feat_dim=None):
    k = _make_sc_ballquery(ncand, radius * radius, feat_dim)
    cx = s[..., 0] + 0.0
    cy = s[..., 1] + 0.0
    cz = s[..., 2] + 0.0
    qx = q[..., 0].reshape(B * M)
    qy = q[..., 1].reshape(B * M)
    qz = q[..., 2].reshape(B * M)
    if feat_dim is None:
        nb = k(cx, cy, cz, qx, qy, qz)
        return nb.reshape(B, M, NS)
    nb, xf = k(cx, cy, cz, qx, qy, qz, tab)
    return nb.reshape(B, M, NS), xf.reshape(B, M, NS, feat_dim)


def _select_body(ncand, r2, q_ref, st_ref, o_ref):
    q = q_ref[0]
    st = st_ref[0]
    q2 = jnp.sum(q * q, axis=1, keepdims=True)
    s2 = jnp.sum(st * st, axis=0, keepdims=True)
    d2 = q2 + s2 - 2.0 * jnp.dot(q, st, preferred_element_type=jnp.float32)
    iota = jax.lax.broadcasted_iota(jnp.int32, (MB, ncand), 1)
    mfull = jnp.min(d2, axis=1, keepdims=True)
    nearest = jnp.min(jnp.where(d2 <= mfull, iota, BIGI), axis=1, keepdims=True)
    cur = jnp.where(d2 <= r2, d2, BIGF)
    for k in range(NS):
        m = jnp.min(cur, axis=1, keepdims=True)
        a = jnp.min(jnp.where(cur <= m, iota, BIGI), axis=1, keepdims=True)
        o_ref[0, :, k:k + 1] = jnp.where(m < BIGF, a, nearest)
        cur = jnp.where(iota == a, BIGF, cur)


def _ball_select(q, st, radius, ncand):
    return pl.pallas_call(
        functools.partial(_select_body, ncand, radius * radius),
        grid=(B, M // MB),
        in_specs=[
            pl.BlockSpec((1, MB, 3), lambda b, i: (b, i, 0)),
            pl.BlockSpec((1, 3, ncand), lambda b, i: (b, 0, 0)),
        ],
        out_specs=pl.BlockSpec((1, MB, NS), lambda b, i: (b, i, 0)),
        out_shape=jax.ShapeDtypeStruct((B, M, NS), jnp.int32),
    )(q, st)


def _sample_idx():
    skey = jax.random.key(42)
    return jnp.stack([
        jax.random.choice(jax.random.fold_in(skey, i), N, shape=(M,), replace=False)
        for i in range(B)
    ])


def kernel(p, f, w_stem, b_stem, w_sa, g_sa, b_sa, w_la, g_la, b_la,
           w_pw1, g_pw1, b_pw1, w_pw2, g_pw2, b_pw2):
    f0 = _stem(jnp.transpose(f, (0, 2, 1)), w_stem.T, b_stem[None, :])

    idx_s = _sample_idx()
    new_p = jax.vmap(lambda pb, ib: pb[ib])(p, idx_s)

    tab = jnp.pad(f0.reshape(B * N, WIDTH), ((0, 0), (0, WIDTH)))
    nb, xf = _sc_ball(new_p, p, R1, N, tab=tab, feat_dim=2 * WIDTH)
    pj = jax.vmap(lambda pb, ib: pb[ib])(p, nb)
    dp = pj - new_p[:, :, None, :]

    wf_pad = jnp.pad(w_sa[:, 3:].T, ((0, WIDTH), (0, 0)))
    f1 = _stage1(dp, xf, w_sa[:, :3].T, wf_pad,
                 g_sa[None, :], b_sa[None, :])

    nb2 = _sc_ball(new_p, new_p, R2, M)
    pj2 = jax.vmap(lambda pb, ib: pb[ib])(new_p, nb2)
    dp2 = pj2 - new_p[:, :, None, :]
    gidx2 = (nb2 + jnp.arange(B, dtype=jnp.int32)[:, None, None] * M).reshape(-1)
    xf2 = _make_sc_rowgather(OUT)(gidx2, f1.reshape(B * M, OUT))
    xf2 = xf2.reshape(B, M, NS, OUT)

    out = _stage2(dp2, xf2, f1, w_la[:, :3].T, w_la[:, 3:].T,
                  g_la[None, :], b_la[None, :],
                  w_pw1.T, g_pw1[None, :], b_pw1[None, :],
                  w_pw2.T, g_pw2[None, :], b_pw2[None, :])
    return jnp.transpose(out, (0, 2, 1))


# speedup vs baseline: 3.9049x; 2.0333x over previous
import functools

import jax
import jax.numpy as jnp
import numpy as np
from jax import lax
from jax.experimental import pallas as pl
from jax.experimental.pallas import tpu as pltpu
from jax.experimental.pallas import tpu_sc as plsc

B = 2
N = 8192
CIN = 4
WIDTH = 64
OUT = 128
STRIDE = 4
NS = 32
R1 = 0.1
R2 = 0.2
MID = 512
M = N // STRIDE

MB = 256
NBLK = 1024


def _stem_body(f_ref, w_ref, b_ref, o_ref):
    x = f_ref[0]
    w = w_ref[...]
    o_ref[0] = jnp.dot(x, w, preferred_element_type=jnp.float32) + b_ref[...]


def _stem(f_t, w_t, b2):
    return pl.pallas_call(
        _stem_body,
        grid=(B, N // NBLK),
        in_specs=[
            pl.BlockSpec((1, NBLK, CIN), lambda b, i: (b, i, 0)),
            pl.BlockSpec((CIN, WIDTH), lambda b, i: (0, 0)),
            pl.BlockSpec((1, WIDTH), lambda b, i: (0, 0)),
        ],
        out_specs=pl.BlockSpec((1, NBLK, WIDTH), lambda b, i: (b, i, 0)),
        out_shape=jax.ShapeDtypeStruct((B, N, WIDTH), jnp.float32),
    )(f_t, w_t, b2)


def _stage1_body(dp_ref, xf_ref, wp_ref, wf_ref, g_ref, b_ref, o_ref):
    dp = dp_ref[...].reshape(MB * NS, 3)
    xf = xf_ref[...].reshape(MB * NS, 2 * WIDTH)
    y = jnp.dot(dp, wp_ref[...], preferred_element_type=jnp.float32)
    y = y + jnp.dot(xf, wf_ref[...], preferred_element_type=jnp.float32)
    y = y * g_ref[...] + b_ref[...]
    y = jnp.maximum(y, 0.0)
    o_ref[0] = jnp.max(y.reshape(MB, NS, OUT), axis=1)


def _stage1(dp, xf, wp, wf, g2, b2):
    return pl.pallas_call(
        _stage1_body,
        grid=(B, M // MB),
        in_specs=[
            pl.BlockSpec((1, MB, NS, 3), lambda b, i: (b, i, 0, 0)),
            pl.BlockSpec((1, MB, NS, 2 * WIDTH), lambda b, i: (b, i, 0, 0)),
            pl.BlockSpec((3, OUT), lambda b, i: (0, 0)),
            pl.BlockSpec((2 * WIDTH, OUT), lambda b, i: (0, 0)),
            pl.BlockSpec((1, OUT), lambda b, i: (0, 0)),
            pl.BlockSpec((1, OUT), lambda b, i: (0, 0)),
        ],
        out_specs=pl.BlockSpec((1, MB, OUT), lambda b, i: (b, i, 0)),
        out_shape=jax.ShapeDtypeStruct((B, M, OUT), jnp.float32),
    )(dp, xf, wp, wf, g2, b2)


def _stage2_body(dp_ref, xf_ref, id_ref, wp_ref, wf_ref, gla_ref, bla_ref,
                 w1_ref, g1_ref, b1_ref, w2_ref, g2_ref, b2_ref, o_ref):
    dp = dp_ref[...].reshape(MB * NS, 3)
    xf = xf_ref[...].reshape(MB * NS, OUT)
    y = jnp.dot(dp, wp_ref[...], preferred_element_type=jnp.float32)
    y = y + jnp.dot(xf, wf_ref[...], preferred_element_type=jnp.float32)
    y = y * gla_ref[...] + bla_ref[...]
    y = jnp.maximum(y, 0.0)
    f2 = jnp.max(y.reshape(MB, NS, OUT), axis=1)
    h = jnp.dot(f2, w1_ref[...], preferred_element_type=jnp.float32)
    h = jnp.maximum(h * g1_ref[...] + b1_ref[...], 0.0)
    o = jnp.dot(h, w2_ref[...], preferred_element_type=jnp.float32)
    o = o * g2_ref[...] + b2_ref[...]
    o_ref[0] = jnp.maximum(o + id_ref[0], 0.0)


def _stage2(dp, xf, ident, wp, wf, gla2, bla2, w1t, g12, b12, w2t, g22, b22):
    return pl.pallas_call(
        _stage2_body,
        grid=(B, M // MB),
        in_specs=[
            pl.BlockSpec((1, MB, NS, 3), lambda b, i: (b, i, 0, 0)),
            pl.BlockSpec((1, MB, NS, OUT), lambda b, i: (b, i, 0, 0)),
            pl.BlockSpec((1, MB, OUT), lambda b, i: (b, i, 0)),
            pl.BlockSpec((3, OUT), lambda b, i: (0, 0)),
            pl.BlockSpec((OUT, OUT), lambda b, i: (0, 0)),
            pl.BlockSpec((1, OUT), lambda b, i: (0, 0)),
            pl.BlockSpec((1, OUT), lambda b, i: (0, 0)),
            pl.BlockSpec((OUT, MID), lambda b, i: (0, 0)),
            pl.BlockSpec((1, MID), lambda b, i: (0, 0)),
            pl.BlockSpec((1, MID), lambda b, i: (0, 0)),
            pl.BlockSpec((MID, OUT), lambda b, i: (0, 0)),
            pl.BlockSpec((1, OUT), lambda b, i: (0, 0)),
            pl.BlockSpec((1, OUT), lambda b, i: (0, 0)),
        ],
        out_specs=pl.BlockSpec((1, MB, OUT), lambda b, i: (b, i, 0)),
        out_shape=jax.ShapeDtypeStruct((B, M, OUT), jnp.float32),
    )(dp, xf, ident, wp, wf, gla2, bla2, w1t, g12, b12, w2t, g22, b22)


BIGF = 1e30
BIGI = 2**30

NTILES = 32
QPT = (B * M) // NTILES
CAP = 512


def _make_sc_ballquery(ncand, r2, feat_dim=None):
    mesh = plsc.VectorSubcoreMesh(core_axis_name="c", subcore_axis_name="s")
    out_type = [jax.ShapeDtypeStruct((B * M * NS,), jnp.int32),
                jax.ShapeDtypeStruct((B * M * NS,), jnp.float32),
                jax.ShapeDtypeStruct((B * M * NS,), jnp.float32),
                jax.ShapeDtypeStruct((B * M * NS,), jnp.float32)]
    scratch = [
        pltpu.VMEM((ncand,), jnp.float32),
        pltpu.VMEM((ncand,), jnp.float32),
        pltpu.VMEM((ncand,), jnp.float32),
        pltpu.VMEM((ncand,), jnp.float32),
        pltpu.VMEM((ncand,), jnp.float32),
        pltpu.VMEM((ncand,), jnp.float32),
        pltpu.VMEM((QPT,), jnp.float32),
        pltpu.VMEM((QPT,), jnp.float32),
        pltpu.VMEM((QPT,), jnp.float32),
        pltpu.VMEM((CAP,), jnp.float32),
        pltpu.VMEM((CAP,), jnp.int32),
        pltpu.VMEM((QPT * NS,), jnp.int32),
        pltpu.VMEM((QPT * NS,), jnp.float32),
        pltpu.VMEM((QPT * NS,), jnp.float32),
        pltpu.VMEM((QPT * NS,), jnp.float32),
        pltpu.VMEM((ncand,), jnp.float32),
    ]
    if feat_dim is not None:
        out_type = out_type + [
            jax.ShapeDtypeStruct((B * M * NS, feat_dim), jnp.float32)]
        scratch += [
            pltpu.VMEM((NS,), jnp.int32),
            pltpu.VMEM((NS, feat_dim), jnp.float32),
            pltpu.SemaphoreType.DMA,
        ]
    out_type = tuple(out_type)

    @functools.partial(
        pl.kernel, mesh=mesh, out_type=out_type, scratch_types=scratch,
        compiler_params=pltpu.CompilerParams(needs_layout_passes=False),
    )
    def k(*refs):
        if feat_dim is not None:
            (cx, cy, cz, qx, qy, qz, tab,
             o_hbm, ox_hbm, oy_hbm, oz_hbm, xf_hbm,
             xs, ys, zs, xr, yr, zr, qxv, qyv, qzv, keyb, idxb,
             outbuf, dpxb, dpyb, dpzb, s2s,
             gidx, rows, dsem) = refs
        else:
            (cx, cy, cz, qx, qy, qz,
             o_hbm, ox_hbm, oy_hbm, oz_hbm,
             xs, ys, zs, xr, yr, zr, qxv, qyv, qzv, keyb, idxb,
             outbuf, dpxb, dpyb, dpzb, s2s) = refs
        cid = lax.axis_index("c")
        sid = lax.axis_index("s")
        wid = sid * 2 + cid
        b = wid // (NTILES // B)
        pltpu.sync_copy(cx.at[b], xs)
        pltpu.sync_copy(cy.at[b], ys)
        pltpu.sync_copy(cz.at[b], zs)
        base_q = wid * QPT
        pltpu.sync_copy(qx.at[pl.ds(base_q, QPT)], qxv)
        pltpu.sync_copy(qy.at[pl.ds(base_q, QPT)], qyv)
        pltpu.sync_copy(qz.at[pl.ds(base_q, QPT)], qzv)
        iota16 = lax.iota(jnp.int32, 16)
        zeros16 = jnp.zeros((16,), jnp.int32)
        bigv = jnp.full((16,), BIGF, jnp.float32)

        def _rbf16(v):
            u = plsc.bitcast(v, jnp.int32)
            u = u + 0x7FFF + ((u >> 16) & 1)
            u = u & jnp.int32(-65536)
            return plsc.bitcast(u, jnp.float32)

        def prep(t, _):
            xv = xs[pl.ds(t * 16, 16)]
            yv = ys[pl.ds(t * 16, 16)]
            zv = zs[pl.ds(t * 16, 16)]
            s2s[pl.ds(t * 16, 16)] = (xv * xv + yv * yv) + zv * zv
            xr[pl.ds(t * 16, 16)] = _rbf16(xv)
            yr[pl.ds(t * 16, 16)] = _rbf16(yv)
            zr[pl.ds(t * 16, 16)] = _rbf16(zv)
            return 0

        lax.fori_loop(0, ncand // 16, prep, 0)

        def per_query(i, _):
            tq = (i // 16) * 16
            lane = zeros16 + (i - tq)
            qxb = qxv[pl.ds(tq, 16)][lane]
            qyb = qyv[pl.ds(tq, 16)][lane]
            qzb = qzv[pl.ds(tq, 16)][lane]
            q2b = (qxb * qxb + qyb * qyb) + qzb * qzb
            qxr = _rbf16(qxb)
            qyr = _rbf16(qyb)
            qzr = _rbf16(qzb)

            def body(t, cntv):
                dot = (xr[pl.ds(t * 16, 16)] * qxr
                       + yr[pl.ds(t * 16, 16)] * qyr
                       + zr[pl.ds(t * 16, 16)] * qzr)
                d2 = (q2b + s2s[pl.ds(t * 16, 16)]) - 2.0 * dot
                mask = d2 <= r2
                pos = plsc.cumsum(mask.astype(jnp.int32)) + cntv - 1
                pos = jnp.minimum(pos, CAP - 1)
                plsc.store_scatter(keyb, [pos], d2, mask=mask)
                plsc.store_scatter(idxb, [pos], iota16 + t * 16, mask=mask)
                return cntv + plsc.all_reduce_population_count(mask)

            cntv = lax.fori_loop(0, ncand // 16, body, zeros16)
            cnt = jnp.minimum(jnp.max(cntv), CAP)
            keyb[pl.ds(jnp.minimum(cnt, CAP - 16), 16)] = bigv
            nv = (cnt + 15) // 16

            def sel(t, carry):
                a0k, a0v, a1k, a1v = carry
                ck = keyb[pl.ds(t * 16, 16)]
                cv = idxb[pl.ds(t * 16, 16)]
                ck, cv = plsc.sort_key_val(ck, cv)
                ckr = lax.rev(ck, (0,))
                cvr = lax.rev(cv, (0,))
                m = a1k <= ckr
                lk = jnp.where(m, a1k, ckr)
                lv = jnp.where(m, a1v, cvr)
                lk, lv = plsc.sort_key_val(lk, lv)
                lkr = lax.rev(lk, (0,))
                lvr = lax.rev(lv, (0,))
                m2 = a0k <= lkr
                n0k = jnp.where(m2, a0k, lkr)
                n0v = jnp.where(m2, a0v, lvr)
                n1k = jnp.where(m2, lkr, a0k)
                n1v = jnp.where(m2, lvr, a0v)
                n0k, n0v = plsc.sort_key_val(n0k, n0v)
                n1k, n1v = plsc.sort_key_val(n1k, n1v)
                return n0k, n0v, n1k, n1v

            a0k, a0v, a1k, a1v = lax.fori_loop(
                0, nv, sel, (bigv, zeros16, bigv, zeros16))
            pad = a0v[zeros16]
            out0 = jnp.where(iota16 < cntv, a0v, pad)
            out1 = jnp.where(iota16 + 16 < cntv, a1v, pad)
            outbuf[pl.ds(i * NS, 16)] = out0
            outbuf[pl.ds(i * NS + 16, 16)] = out1
            dpxb[pl.ds(i * NS, 16)] = plsc.load_gather(xs, [out0]) - qxb
            dpxb[pl.ds(i * NS + 16, 16)] = plsc.load_gather(xs, [out1]) - qxb
            dpyb[pl.ds(i * NS, 16)] = plsc.load_gather(ys, [out0]) - qyb
            dpyb[pl.ds(i * NS + 16, 16)] = plsc.load_gather(ys, [out1]) - qyb
            dpzb[pl.ds(i * NS, 16)] = plsc.load_gather(zs, [out0]) - qzb
            dpzb[pl.ds(i * NS + 16, 16)] = plsc.load_gather(zs, [out1]) - qzb
            if feat_dim is not None:
                gidx[pl.ds(0, 16)] = out0 + b * ncand
                gidx[pl.ds(16, 16)] = out1 + b * ncand
                pltpu.async_copy(tab.at[gidx], rows, dsem).wait()
                pltpu.sync_copy(
                    rows, xf_hbm.at[pl.ds((base_q + i) * NS, NS)])
            return 0

        lax.fori_loop(0, QPT, per_query, 0)
        pltpu.sync_copy(outbuf, o_hbm.at[pl.ds(base_q * NS, QPT * NS)])
        pltpu.sync_copy(dpxb, ox_hbm.at[pl.ds(base_q * NS, QPT * NS)])
        pltpu.sync_copy(dpyb, oy_hbm.at[pl.ds(base_q * NS, QPT * NS)])
        pltpu.sync_copy(dpzb, oz_hbm.at[pl.ds(base_q * NS, QPT * NS)])

    return k


def _make_sc_rowgather(feat_dim):
    mesh = plsc.VectorSubcoreMesh(core_axis_name="c", subcore_axis_name="s")
    NIDX = B * M * NS
    IPT = NIDX // NTILES

    @functools.partial(
        pl.kernel, mesh=mesh,
        out_type=jax.ShapeDtypeStruct((NIDX, feat_dim), jnp.float32),
        scratch_types=[
            pltpu.VMEM((IPT,), jnp.int32),
            pltpu.VMEM((NS, feat_dim), jnp.float32),
            pltpu.SemaphoreType.DMA,
        ],
        compiler_params=pltpu.CompilerParams(needs_layout_passes=False),
    )
    def k(idx_hbm, tab, o_hbm, idxv, rows, dsem):
        cid = lax.axis_index("c")
        sid = lax.axis_index("s")
        wid = sid * 2 + cid
        base = wid * IPT
        pltpu.sync_copy(idx_hbm.at[pl.ds(base, IPT)], idxv)

        def per_chunk(i, _):
            pltpu.async_copy(
                tab.at[idxv.at[pl.ds(i * NS, NS)]], rows, dsem).wait()
            pltpu.sync_copy(rows, o_hbm.at[pl.ds(base + i * NS, NS)])
            return 0

        lax.fori_loop(0, IPT // NS, per_chunk, 0)

    return k


def _sc_ball(q, s, radius, ncand, tab=None, feat_dim=None):
    k = _make_sc_ballquery(ncand, radius * radius, feat_dim)
    cx = s[..., 0] + 0.0
    cy = s[..., 1] + 0.0
    cz = s[..., 2] + 0.0
    qx = q[..., 0].reshape(B * M)
    qy = q[..., 1].reshape(B * M)
    qz = q[..., 2].reshape(B * M)
    if feat_dim is None:
        nb, ox, oy, oz = k(cx, cy, cz, qx, qy, qz)
        dp = jnp.stack([ox, oy, oz], axis=-1).reshape(B, M, NS, 3)
        return nb.reshape(B, M, NS), dp
    nb, ox, oy, oz, xf = k(cx, cy, cz, qx, qy, qz, tab)
    dp = jnp.stack([ox, oy, oz], axis=-1).reshape(B, M, NS, 3)
    return nb.reshape(B, M, NS), dp, xf.reshape(B, M, NS, feat_dim)


def _select_body(ncand, r2, q_ref, st_ref, o_ref):
    q = q_ref[0]
    st = st_ref[0]
    q2 = jnp.sum(q * q, axis=1, keepdims=True)
    s2 = jnp.sum(st * st, axis=0, keepdims=True)
    d2 = q2 + s2 - 2.0 * jnp.dot(q, st, preferred_element_type=jnp.float32)
    iota = jax.lax.broadcasted_iota(jnp.int32, (MB, ncand), 1)
    mfull = jnp.min(d2, axis=1, keepdims=True)
    nearest = jnp.min(jnp.where(d2 <= mfull, iota, BIGI), axis=1, keepdims=True)
    cur = jnp.where(d2 <= r2, d2, BIGF)
    for k in range(NS):
        m = jnp.min(cur, axis=1, keepdims=True)
        a = jnp.min(jnp.where(cur <= m, iota, BIGI), axis=1, keepdims=True)
        o_ref[0, :, k:k + 1] = jnp.where(m < BIGF, a, nearest)
        cur = jnp.where(iota == a, BIGF, cur)


def _ball_select(q, st, radius, ncand):
    return pl.pallas_call(
        functools.partial(_select_body, ncand, radius * radius),
        grid=(B, M // MB),
        in_specs=[
            pl.BlockSpec((1, MB, 3), lambda b, i: (b, i, 0)),
            pl.BlockSpec((1, 3, ncand), lambda b, i: (b, 0, 0)),
        ],
        out_specs=pl.BlockSpec((1, MB, NS), lambda b, i: (b, i, 0)),
        out_shape=jax.ShapeDtypeStruct((B, M, NS), jnp.int32),
    )(q, st)


def _sample_idx():
    skey = jax.random.key(42)
    return jnp.stack([
        jax.random.choice(jax.random.fold_in(skey, i), N, shape=(M,), replace=False)
        for i in range(B)
    ])


def kernel(p, f, w_stem, b_stem, w_sa, g_sa, b_sa, w_la, g_la, b_la,
           w_pw1, g_pw1, b_pw1, w_pw2, g_pw2, b_pw2):
    f0 = _stem(jnp.transpose(f, (0, 2, 1)), w_stem.T, b_stem[None, :])

    idx_s = _sample_idx()
    new_p = jax.vmap(lambda pb, ib: pb[ib])(p, idx_s)

    tab = jnp.pad(f0.reshape(B * N, WIDTH), ((0, 0), (0, WIDTH)))
    nb, dp, xf = _sc_ball(new_p, p, R1, N, tab=tab, feat_dim=2 * WIDTH)

    wf_pad = jnp.pad(w_sa[:, 3:].T, ((0, WIDTH), (0, 0)))
    f1 = _stage1(dp, xf, w_sa[:, :3].T, wf_pad,
                 g_sa[None, :], b_sa[None, :])

    nb2, dp2 = _sc_ball(new_p, new_p, R2, M)
    gidx2 = (nb2 + jnp.arange(B, dtype=jnp.int32)[:, None, None] * M).reshape(-1)
    xf2 = _make_sc_rowgather(OUT)(gidx2, f1.reshape(B * M, OUT))
    xf2 = xf2.reshape(B, M, NS, OUT)

    out = _stage2(dp2, xf2, f1, w_la[:, :3].T, w_la[:, 3:].T,
                  g_la[None, :], b_la[None, :],
                  w_pw1.T, g_pw1[None, :], b_pw1[None, :],
                  w_pw2.T, g_pw2[None, :], b_pw2[None, :])
    return jnp.transpose(out, (0, 2, 1))
